# baseline (speedup 1.0000x reference)
import functools
import math

import jax
import jax.numpy as jnp
from jax import lax
from jax.experimental import pallas as pl
from jax.experimental.pallas import tpu as pltpu
from jax.experimental.pallas import tpu_sc as plsc

_GAMMA = 12.0
_EPS = 2.0
_EMB_DIM = 64
_EMB_RANGE = (_GAMMA + _EPS) / _EMB_DIM
_PI = math.pi

_B = 16384
_NC = 2
_NS = 16
_NW = _NC * _NS
_N_PER_W = _B // _NW
_CHUNK = 64
_NCHUNK = _N_PER_W // _CHUNK


def _trig_body(rel_ref, trig_ref):
    ph = rel_ref[...] * (_PI / _EMB_RANGE)
    trig_ref[:, :_EMB_DIM] = jnp.cos(ph)
    trig_ref[:, _EMB_DIM:] = jnp.sin(ph)


def _rsqrt_newton(x):
    i = lax.bitcast_convert_type(x, jnp.int32)
    i = 0x5F3759DF - lax.shift_right_arithmetic(i, 1)
    y = lax.bitcast_convert_type(i, jnp.float32)
    for _ in range(2):
        y = y * (1.5 - 0.5 * x * y * y)
    return y


def _sc_score(sample, ent, trig_t):
    mesh = plsc.VectorSubcoreMesh(core_axis_name="c", subcore_axis_name="s")

    buf = lambda shape, dt=jnp.float32: pltpu.VMEM(shape, dt)

    @functools.partial(
        pl.kernel,
        out_type=jax.ShapeDtypeStruct((_B,), jnp.float32),
        mesh=mesh,
        compiler_params=pltpu.CompilerParams(needs_layout_passes=False),
        scratch_types=[
            buf((_N_PER_W, 3), jnp.int32),
            buf((2 * _N_PER_W,), jnp.int32),
            buf((_N_PER_W,), jnp.int32),
            [buf((2 * _CHUNK, 128)) for _ in range(2)],
            [buf((_CHUNK, 128)) for _ in range(2)],
            buf((_N_PER_W,)),
            pltpu.SemaphoreType.DMA,
            pltpu.SemaphoreType.DMA,
        ],
    )
    def sc_kernel(samp_hbm, ent_hbm, trig_hbm, out_hbm, samp_v, htid_v,
                  rid_v, ht_v, trig_v, out_v, sem0, sem1):
        wid = lax.axis_index("s") * _NC + lax.axis_index("c")
        base = wid * _N_PER_W
        lane = lax.iota(jnp.int32, 16)
        col0 = jnp.zeros((16,), jnp.int32)
        col1 = col0 + 1
        col2 = col0 + 2
        sems = (sem0, sem1)

        pltpu.sync_copy(samp_hbm.at[pl.ds(base, _N_PER_W)], samp_v)
        for j in range(_N_PER_W // 16):
            rows = j * 16 + lane
            c, g = divmod(j, _CHUNK // 16)
            hslot = pl.ds(c * 2 * _CHUNK + g * 16, 16)
            tslot = pl.ds(c * 2 * _CHUNK + _CHUNK + g * 16, 16)
            htid_v[hslot] = plsc.load_gather(samp_v, [rows, col0])
            htid_v[tslot] = plsc.load_gather(samp_v, [rows, col2])
            rid_v[pl.ds(j * 16, 16)] = plsc.load_gather(samp_v, [rows, col1])

        def issue(c, b):
            return (
                pltpu.async_copy(
                    ent_hbm.at[htid_v.at[pl.ds(c * 2 * _CHUNK, 2 * _CHUNK)]],
                    ht_v[b], sems[b]),
                pltpu.async_copy(
                    trig_hbm.at[rid_v.at[pl.ds(c * _CHUNK, _CHUNK)]],
                    trig_v[b], sems[b]),
            )

        def compute(c, b):
            ht, trig = ht_v[b], trig_v[b]

            def group_body(g, _):
                def sample_body(j, vec):
                    s = g * 16 + j
                    acc = jnp.zeros((16,), jnp.float32)
                    for k in range(4):
                        re_h = ht[s, pl.ds(k * 16, 16)]
                        im_h = ht[s, pl.ds(64 + k * 16, 16)]
                        re_t = ht[_CHUNK + s, pl.ds(k * 16, 16)]
                        im_t = ht[_CHUNK + s, pl.ds(64 + k * 16, 16)]
                        re_r = trig[s, pl.ds(k * 16, 16)]
                        im_r = trig[s, pl.ds(64 + k * 16, 16)]
                        a = re_h * re_r - im_h * im_r - re_t
                        bb = re_h * im_r + im_h * re_r - im_t
                        x = a * a + bb * bb
                        x = jnp.maximum(x, 1e-12)
                        acc = acc + x * _rsqrt_newton(x)
                    total = _GAMMA - jnp.sum(acc)
                    return jnp.where(lane == j, total, vec)

                vec = lax.fori_loop(0, 16, sample_body,
                                    jnp.zeros((16,), jnp.float32),
                                    unroll=8)
                out_v[pl.ds(c * _CHUNK + g * 16, 16)] = vec
                return _

            lax.fori_loop(0, _CHUNK // 16, group_body, 0)

        handles = issue(0, 0)
        for c in range(_NCHUNK):
            nxt = None
            if c + 1 < _NCHUNK:
                nxt = issue(c + 1, (c + 1) % 2)
            for h in handles:
                h.wait()
            compute(c, c % 2)
            handles = nxt
        pltpu.sync_copy(out_v, out_hbm.at[pl.ds(base, _N_PER_W)])

    return sc_kernel(sample, ent, trig_t)


def kernel(sample, entity_embedding, relation_embedding):
    trig_t = pl.pallas_call(
        _trig_body,
        out_shape=jax.ShapeDtypeStruct(
            (relation_embedding.shape[0], 2 * _EMB_DIM), jnp.float32),
    )(relation_embedding)
    score = _sc_score(sample, entity_embedding, trig_t)
    return score.reshape(_B, 1)


# speedup vs baseline: 1.0224x; 1.0224x over previous
import functools
import math

import jax
import jax.numpy as jnp
from jax import lax
from jax.experimental import pallas as pl
from jax.experimental.pallas import tpu as pltpu
from jax.experimental.pallas import tpu_sc as plsc

_GAMMA = 12.0
_EPS = 2.0
_EMB_DIM = 64
_EMB_RANGE = (_GAMMA + _EPS) / _EMB_DIM
_PI = math.pi

_B = 16384
_NC = 2
_NS = 16
_NW = _NC * _NS
_N_PER_W = _B // _NW
_CHUNK = 64
_NCHUNK = _N_PER_W // _CHUNK


def _trig_body(rel_ref, trig_ref):
    ph = rel_ref[...] * (_PI / _EMB_RANGE)
    trig_ref[:, :_EMB_DIM] = jnp.cos(ph)
    trig_ref[:, _EMB_DIM:] = jnp.sin(ph)


def _rsqrt_newton(x):
    i = lax.bitcast_convert_type(x, jnp.int32)
    i = 0x5F3759DF - lax.shift_right_arithmetic(i, 1)
    y = lax.bitcast_convert_type(i, jnp.float32)
    for _ in range(2):
        y = y * (1.5 - 0.5 * x * y * y)
    return y


def _sc_score(sample, ent, trig_t):
    mesh = plsc.VectorSubcoreMesh(core_axis_name="c", subcore_axis_name="s")

    buf = lambda shape, dt=jnp.float32: pltpu.VMEM(shape, dt)

    @functools.partial(
        pl.kernel,
        out_type=jax.ShapeDtypeStruct((_B,), jnp.float32),
        mesh=mesh,
        compiler_params=pltpu.CompilerParams(needs_layout_passes=False),
        scratch_types=[
            buf((_N_PER_W, 3), jnp.int32),
            buf((2 * _N_PER_W,), jnp.int32),
            buf((_N_PER_W,), jnp.int32),
            [buf((2 * _CHUNK, 128)) for _ in range(2)],
            [buf((_CHUNK, 128)) for _ in range(2)],
            buf((_N_PER_W,)),
            pltpu.SemaphoreType.DMA,
            pltpu.SemaphoreType.DMA,
        ],
    )
    def sc_kernel(samp_hbm, ent_hbm, trig_hbm, out_hbm, samp_v, htid_v,
                  rid_v, ht_v, trig_v, out_v, sem0, sem1):
        wid = lax.axis_index("s") * _NC + lax.axis_index("c")
        base = wid * _N_PER_W
        lane = lax.iota(jnp.int32, 16)
        col0 = jnp.zeros((16,), jnp.int32)
        col1 = col0 + 1
        col2 = col0 + 2
        sems = (sem0, sem1)

        pltpu.sync_copy(samp_hbm.at[pl.ds(base, _N_PER_W)], samp_v)
        for j in range(_N_PER_W // 16):
            rows = j * 16 + lane
            c, g = divmod(j, _CHUNK // 16)
            hslot = pl.ds(c * 2 * _CHUNK + g * 16, 16)
            tslot = pl.ds(c * 2 * _CHUNK + _CHUNK + g * 16, 16)
            htid_v[hslot] = plsc.load_gather(samp_v, [rows, col0])
            htid_v[tslot] = plsc.load_gather(samp_v, [rows, col2])
            rid_v[pl.ds(j * 16, 16)] = plsc.load_gather(samp_v, [rows, col1])

        def issue(c, b):
            return (
                pltpu.async_copy(
                    ent_hbm.at[htid_v.at[pl.ds(c * 2 * _CHUNK, 2 * _CHUNK)]],
                    ht_v[b], sems[b]),
                pltpu.async_copy(
                    trig_hbm.at[rid_v.at[pl.ds(c * _CHUNK, _CHUNK)]],
                    trig_v[b], sems[b]),
            )

        def compute(c, b):
            ht, trig = ht_v[b], trig_v[b]

            def group_body(g, _):
                def sample_body(j, vec):
                    s = g * 16 + j
                    acc = jnp.zeros((16,), jnp.float32)
                    for k in range(4):
                        re_h = ht[s, pl.ds(k * 16, 16)]
                        im_h = ht[s, pl.ds(64 + k * 16, 16)]
                        re_t = ht[_CHUNK + s, pl.ds(k * 16, 16)]
                        im_t = ht[_CHUNK + s, pl.ds(64 + k * 16, 16)]
                        re_r = trig[s, pl.ds(k * 16, 16)]
                        im_r = trig[s, pl.ds(64 + k * 16, 16)]
                        a = re_h * re_r - im_h * im_r - re_t
                        bb = re_h * im_r + im_h * re_r - im_t
                        x = a * a + bb * bb
                        x = jnp.maximum(x, 1e-12)
                        acc = acc + x * _rsqrt_newton(x)
                    total = _GAMMA - jnp.sum(acc)
                    return jnp.where(lane == j, total, vec)

                vec = lax.fori_loop(0, 16, sample_body,
                                    jnp.zeros((16,), jnp.float32),
                                    unroll=2)
                out_v[pl.ds(c * _CHUNK + g * 16, 16)] = vec
                return _

            lax.fori_loop(0, _CHUNK // 16, group_body, 0)

        handles = issue(0, 0)
        for c in range(_NCHUNK):
            nxt = None
            if c + 1 < _NCHUNK:
                nxt = issue(c + 1, (c + 1) % 2)
            for h in handles:
                h.wait()
            compute(c, c % 2)
            handles = nxt
        pltpu.sync_copy(out_v, out_hbm.at[pl.ds(base, _N_PER_W)])

    return sc_kernel(sample, ent, trig_t)


def kernel(sample, entity_embedding, relation_embedding):
    trig_t = pl.pallas_call(
        _trig_body,
        out_shape=jax.ShapeDtypeStruct(
            (relation_embedding.shape[0], 2 * _EMB_DIM), jnp.float32),
    )(relation_embedding)
    score = _sc_score(sample, entity_embedding, trig_t)
    return score.reshape(_B, 1)


# speedup vs baseline: 1.0543x; 1.0312x over previous
import functools
import math

import jax
import jax.numpy as jnp
from jax import lax
from jax.experimental import pallas as pl
from jax.experimental.pallas import tpu as pltpu
from jax.experimental.pallas import tpu_sc as plsc

_GAMMA = 12.0
_EPS = 2.0
_EMB_DIM = 64
_EMB_RANGE = (_GAMMA + _EPS) / _EMB_DIM
_PI = math.pi

_B = 16384
_NC = 2
_NS = 16
_NW = _NC * _NS
_N_PER_W = _B // _NW
_CHUNK = 64
_NCHUNK = _N_PER_W // _CHUNK


def _trig_body(rel_ref, trig_ref):
    ph = rel_ref[...] * (_PI / _EMB_RANGE)
    trig_ref[:, :_EMB_DIM] = jnp.cos(ph)
    trig_ref[:, _EMB_DIM:] = jnp.sin(ph)


def _rsqrt_newton(x):
    i = lax.bitcast_convert_type(x, jnp.int32)
    i = 0x5F3759DF - lax.shift_right_arithmetic(i, 1)
    y = lax.bitcast_convert_type(i, jnp.float32)
    for _ in range(2):
        y = y * (1.5 - 0.5 * x * y * y)
    return y


def _sc_score(sample, ent, trig_t):
    mesh = plsc.VectorSubcoreMesh(core_axis_name="c", subcore_axis_name="s")

    buf = lambda shape, dt=jnp.float32: pltpu.VMEM(shape, dt)

    @functools.partial(
        pl.kernel,
        out_type=jax.ShapeDtypeStruct((_B,), jnp.float32),
        mesh=mesh,
        compiler_params=pltpu.CompilerParams(needs_layout_passes=False),
        scratch_types=[
            buf((_N_PER_W, 3), jnp.int32),
            buf((2 * _N_PER_W,), jnp.int32),
            buf((_N_PER_W,), jnp.int32),
            [buf((2 * _CHUNK, 128)) for _ in range(2)],
            [buf((_CHUNK, 128)) for _ in range(2)],
            buf((_N_PER_W,)),
            pltpu.SemaphoreType.DMA,
            pltpu.SemaphoreType.DMA,
        ],
    )
    def sc_kernel(samp_hbm, ent_hbm, trig_hbm, out_hbm, samp_v, htid_v,
                  rid_v, ht_v, trig_v, out_v, sem0, sem1):
        wid = lax.axis_index("s") * _NC + lax.axis_index("c")
        base = wid * _N_PER_W
        lane = lax.iota(jnp.int32, 16)
        col0 = jnp.zeros((16,), jnp.int32)
        col1 = col0 + 1
        col2 = col0 + 2
        sems = (sem0, sem1)

        pltpu.sync_copy(samp_hbm.at[pl.ds(base, _N_PER_W)], samp_v)
        gpc = _CHUNK // 16

        def extract_body(j, _):
            rows = j * 16 + lane
            c = j // gpc
            g = j - c * gpc
            hslot = pl.ds(c * 2 * _CHUNK + g * 16, 16)
            tslot = pl.ds(c * 2 * _CHUNK + _CHUNK + g * 16, 16)
            htid_v[hslot] = plsc.load_gather(samp_v, [rows, col0])
            htid_v[tslot] = plsc.load_gather(samp_v, [rows, col2])
            rid_v[pl.ds(j * 16, 16)] = plsc.load_gather(samp_v, [rows, col1])
            return _

        lax.fori_loop(0, _N_PER_W // 16, extract_body, 0)

        def issue(c, b):
            c = jnp.minimum(c, _NCHUNK - 1)
            pltpu.async_copy(
                ent_hbm.at[htid_v.at[pl.ds(c * 2 * _CHUNK, 2 * _CHUNK)]],
                ht_v[b], sems[b])
            pltpu.async_copy(
                trig_hbm.at[rid_v.at[pl.ds(c * _CHUNK, _CHUNK)]],
                trig_v[b], sems[b])

        def drain(b):
            pltpu.make_async_copy(
                ent_hbm.at[htid_v.at[pl.ds(0, 2 * _CHUNK)]],
                ht_v[b], sems[b]).wait()
            pltpu.make_async_copy(
                trig_hbm.at[rid_v.at[pl.ds(0, _CHUNK)]],
                trig_v[b], sems[b]).wait()

        def compute(c, b):
            ht, trig = ht_v[b], trig_v[b]

            def group_body(g, _):
                def sample_body(j, vec):
                    s = g * 16 + j
                    acc = jnp.zeros((16,), jnp.float32)
                    for k in range(4):
                        re_h = ht[s, pl.ds(k * 16, 16)]
                        im_h = ht[s, pl.ds(64 + k * 16, 16)]
                        re_t = ht[_CHUNK + s, pl.ds(k * 16, 16)]
                        im_t = ht[_CHUNK + s, pl.ds(64 + k * 16, 16)]
                        re_r = trig[s, pl.ds(k * 16, 16)]
                        im_r = trig[s, pl.ds(64 + k * 16, 16)]
                        a = re_h * re_r - im_h * im_r - re_t
                        bb = re_h * im_r + im_h * re_r - im_t
                        x = a * a + bb * bb
                        x = jnp.maximum(x, 1e-12)
                        acc = acc + x * _rsqrt_newton(x)
                    total = _GAMMA - jnp.sum(acc)
                    return jnp.where(lane == j, total, vec)

                vec = lax.fori_loop(0, 16, sample_body,
                                    jnp.zeros((16,), jnp.float32),
                                    unroll=2)
                out_v[pl.ds(c * _CHUNK + g * 16, 16)] = vec
                return _

            lax.fori_loop(0, _CHUNK // 16, group_body, 0)

        issue(0, 0)

        def pair_body(p, _):
            c0 = 2 * p
            issue(c0 + 1, 1)
            drain(0)
            compute(c0, 0)
            issue(c0 + 2, 0)
            drain(1)
            compute(c0 + 1, 1)
            return _

        lax.fori_loop(0, _NCHUNK // 2, pair_body, 0)
        drain(0)
        pltpu.sync_copy(out_v, out_hbm.at[pl.ds(base, _N_PER_W)])

    return sc_kernel(sample, ent, trig_t)


def kernel(sample, entity_embedding, relation_embedding):
    trig_t = pl.pallas_call(
        _trig_body,
        out_shape=jax.ShapeDtypeStruct(
            (relation_embedding.shape[0], 2 * _EMB_DIM), jnp.float32),
    )(relation_embedding)
    score = _sc_score(sample, entity_embedding, trig_t)
    return score.reshape(_B, 1)
